# baseline (speedup 1.0000x reference)
import functools

import jax
import jax.numpy as jnp
from jax import lax
from jax.experimental import pallas as pl
from jax.experimental.pallas import tpu as pltpu
from jax.experimental.pallas import tpu_sc as plsc

N_NODES = 10000
D_FEAT = 128
NC, NS = 2, 16
NW = NC * NS
EB = 128
ACC_ROWS = 10112
STRIPE = ACC_ROWS // NS
DUMMY = N_NODES

_MESH = plsc.VectorSubcoreMesh(core_axis_name="c", subcore_axis_name="s",
                               num_cores=NC, num_subcores=NS)


def _degree_body(rows_hbm, ones_hbm, zeros_hbm, out_hbm, ridx, ones_v, acc):
    nb = ridx.shape[0]
    cid = lax.axis_index("c")
    sid = lax.axis_index("s")
    wid = sid * NC + cid
    pltpu.sync_copy(zeros_hbm, acc.at[pl.ds(sid * STRIPE, STRIPE)])
    pltpu.sync_copy(rows_hbm.at[wid], ridx)
    pltpu.sync_copy(ones_hbm, ones_v)
    plsc.subcore_barrier()

    def step(b, carry):
        pltpu.sync_copy(ones_v, acc.at[ridx.at[b]], add=True)
        return carry

    lax.fori_loop(0, nb, step, 0)
    plsc.subcore_barrier()
    pltpu.sync_copy(acc.at[pl.ds(sid * STRIPE, STRIPE)],
                    out_hbm.at[cid, pl.ds(sid * STRIPE, STRIPE)])


def _make_degree(nb):
    return pl.kernel(
        _degree_body,
        out_type=jax.ShapeDtypeStruct((NC, ACC_ROWS, D_FEAT), jnp.float32),
        mesh=_MESH,
        scratch_types=[
            pltpu.VMEM((nb, EB), jnp.int32),
            pltpu.VMEM((EB, D_FEAT), jnp.float32),
            pltpu.VMEM_SHARED((ACC_ROWS, D_FEAT), jnp.float32),
        ],
    )


NBUF = 2
SB = 8


def _spmm_body(nb0, nb1, g_hbm, rows_hbm, cols_hbm, zeros_hbm, out_hbm,
               rring, cring, bufs, acc, gsems, isem):
    cid = lax.axis_index("c")
    sid = lax.axis_index("s")
    wid = sid * NC + cid
    nb = jnp.where(cid == 0, nb0, nb1)
    ng = jnp.where(cid == 0, nb0 // SB, nb1 // SB)
    pltpu.sync_copy(zeros_hbm, acc.at[pl.ds(sid * STRIPE, STRIPE)])

    def stage(grp, slot, sem):
        off = pl.multiple_of(grp * SB, SB)
        roff = pl.multiple_of(slot * SB, SB)
        pltpu.async_copy(rows_hbm.at[wid, pl.ds(off, SB)],
                         rring.at[pl.ds(roff, SB)], sem)
        pltpu.async_copy(cols_hbm.at[wid, pl.ds(off, SB)],
                         cring.at[pl.ds(roff, SB)], sem)

    def stage_wait(grp, slot, sem):
        off = pl.multiple_of(grp * SB, SB)
        roff = pl.multiple_of(slot * SB, SB)
        pltpu.make_async_copy(rows_hbm.at[wid, pl.ds(off, SB)],
                              rring.at[pl.ds(roff, SB)], sem).wait()
        pltpu.make_async_copy(cols_hbm.at[wid, pl.ds(off, SB)],
                              cring.at[pl.ds(roff, SB)], sem).wait()

    stage(0, 0, isem)
    stage_wait(0, 0, isem)
    plsc.subcore_barrier()

    @pl.when(ng > 1)
    def _():
        stage(1, 1, isem)

    for j in range(NBUF):
        pltpu.async_copy(g_hbm.at[cring.at[j]], bufs[j], gsems[j])

    def group(i, carry):
        s = lax.rem(i, 2) * SB
        w = lax.rem(i + 1, 2) * SB

        @pl.when(i + 1 < ng)
        def _():
            stage_wait(i + 1, lax.rem(i + 1, 2), isem)

        for j in range(SB):
            b = SB * i + j
            k = j % NBUF
            nbase = s if j + NBUF < SB else w
            nj = (j + NBUF) % SB
            pltpu.make_async_copy(g_hbm.at[cring.at[s + j]], bufs[k],
                                  gsems[k]).wait()
            pltpu.sync_copy(bufs[k], acc.at[rring.at[s + j]], add=True)

            @pl.when(b + NBUF < nb)
            def _():
                pltpu.async_copy(g_hbm.at[cring.at[nbase + nj]], bufs[k],
                                 gsems[k])

        @pl.when(i + 2 < ng)
        def _():
            stage(i + 2, lax.rem(i, 2), isem)

        return carry

    lax.fori_loop(0, ng, group, 0)
    plsc.subcore_barrier()
    pltpu.sync_copy(acc.at[pl.ds(sid * STRIPE, STRIPE)],
                    out_hbm.at[cid, pl.ds(sid * STRIPE, STRIPE)])


def _make_spmm(nb0, nb1):
    return pl.kernel(
        functools.partial(_spmm_body, nb0, nb1),
        out_type=jax.ShapeDtypeStruct((NC, ACC_ROWS, D_FEAT), jnp.float32),
        mesh=_MESH,
        scratch_types=[
            pltpu.VMEM((2 * SB, EB), jnp.int32),
            pltpu.VMEM((2 * SB, EB), jnp.int32),
            [pltpu.VMEM((EB, D_FEAT), jnp.float32) for _ in range(NBUF)],
            pltpu.VMEM_SHARED((ACC_ROWS, D_FEAT), jnp.float32),
            [pltpu.SemaphoreType.DMA for _ in range(NBUF)],
            pltpu.SemaphoreType.DMA,
        ],
    )


def _prep_body(x_ref, degp_ref, g0_ref):
    x = x_ref[...]
    n = x.shape[0]
    mean = jnp.mean(x, axis=0, keepdims=True)
    xc = x - mean
    var = jnp.sum(xc * xc, axis=0, keepdims=True) / (n - 1)
    rstd = jnp.where(var > 0.0, lax.rsqrt(var), 1.0)
    deg = degp_ref[0, :N_NODES, 0:1] + degp_ref[1, :N_NODES, 0:1] + 1.0
    s = lax.rsqrt(deg)
    g0_ref[...] = xc * rstd * s


def _combine_body(zp_ref, g_ref, degp_ref, out_ref, *, last_hop):
    deg = degp_ref[0, :N_NODES, 0:1] + degp_ref[1, :N_NODES, 0:1] + 1.0
    scale = lax.rsqrt(deg) if last_hop else 1.0 / deg
    z = zp_ref[0, :N_NODES, :] + zp_ref[1, :N_NODES, :] + g_ref[...]
    out_ref[...] = z * scale


SHARE0 = 1.0


def _pack(arr, fill, nb_c, nb_max):
    pad = NS * nb_c * EB - arr.shape[0]
    p = jnp.concatenate([arr, jnp.full((pad,), fill, jnp.int32)])
    p = p.reshape(NS, nb_c, EB)
    if nb_max > nb_c:
        p = jnp.pad(p, ((0, 0), (0, nb_max - nb_c), (0, 0)),
                    constant_values=fill)
    return p


def kernel(x, edge_index):
    e = edge_index.shape[1]
    row = edge_index[0].astype(jnp.int32)
    col = edge_index[1].astype(jnp.int32)
    onesd = jnp.ones((EB, D_FEAT), jnp.float32)
    zerosd = jnp.zeros((STRIPE, D_FEAT), jnp.float32)

    nbd = -(-e // (NW * EB))
    padd = NW * nbd * EB - e
    rows_d = jnp.concatenate(
        [row, jnp.full((padd,), DUMMY, jnp.int32)]).reshape(NW, nbd, EB)
    degp = _make_degree(nbd)(rows_d, onesd, zerosd)

    tb = -(-e // EB)
    nb0 = max(SB, ((int(tb * SHARE0 / NS) + SB // 2) // SB) * SB)
    e0 = min(NS * nb0 * EB, e)
    nb1 = -(-(e - e0) // (NS * EB)) if e > e0 else SB
    nb1 = max(SB, -(-nb1 // SB) * SB)
    nb_max = max(nb0, nb1)
    rows_p = jnp.stack(
        [_pack(row[:e0], DUMMY, nb0, nb_max),
         _pack(row[e0:], DUMMY, nb1, nb_max)],
        axis=1).reshape(NW, nb_max, EB)
    cols_p = jnp.stack(
        [_pack(col[:e0], 0, nb0, nb_max),
         _pack(col[e0:], 0, nb1, nb_max)],
        axis=1).reshape(NW, nb_max, EB)

    prep = pl.pallas_call(
        _prep_body,
        out_shape=jax.ShapeDtypeStruct((N_NODES, D_FEAT), jnp.float32),
    )
    g0 = prep(x, degp)

    spmm = _make_spmm(nb0, nb1)
    combine1 = pl.pallas_call(
        functools.partial(_combine_body, last_hop=False),
        out_shape=jax.ShapeDtypeStruct((N_NODES, D_FEAT), jnp.float32),
    )
    combine2 = pl.pallas_call(
        functools.partial(_combine_body, last_hop=True),
        out_shape=jax.ShapeDtypeStruct((N_NODES, D_FEAT), jnp.float32),
    )

    zp1 = spmm(g0, rows_p, cols_p, zerosd)
    g1 = combine1(zp1, g0, degp)
    zp2 = spmm(g1, rows_p, cols_p, zerosd)
    return combine2(zp2, g1, degp)


# speedup vs baseline: 6.4391x; 6.4391x over previous
import functools

import jax
import jax.numpy as jnp
from jax import lax
from jax.experimental import pallas as pl
from jax.experimental.pallas import tpu as pltpu
from jax.experimental.pallas import tpu_sc as plsc

N_NODES = 10000
D_FEAT = 128
NC, NS = 2, 16
NW = NC * NS
EB = 128
ACC_ROWS = 10112
STRIPE = ACC_ROWS // NS
DUMMY = N_NODES

_MESH = plsc.VectorSubcoreMesh(core_axis_name="c", subcore_axis_name="s",
                               num_cores=NC, num_subcores=NS)


def _degree_body(rows_hbm, ones_hbm, zeros_hbm, out_hbm, ridx, ones_v, acc):
    nb = ridx.shape[0]
    cid = lax.axis_index("c")
    sid = lax.axis_index("s")
    wid = sid * NC + cid
    pltpu.sync_copy(zeros_hbm, acc.at[pl.ds(sid * STRIPE, STRIPE)])
    pltpu.sync_copy(rows_hbm.at[wid], ridx)
    pltpu.sync_copy(ones_hbm, ones_v)
    plsc.subcore_barrier()

    def step(b, carry):
        pltpu.sync_copy(ones_v, acc.at[ridx.at[b]], add=True)
        return carry

    lax.fori_loop(0, nb, step, 0)
    plsc.subcore_barrier()
    pltpu.sync_copy(acc.at[pl.ds(sid * STRIPE, STRIPE)],
                    out_hbm.at[cid, pl.ds(sid * STRIPE, STRIPE)])


def _make_degree(nb):
    return pl.kernel(
        _degree_body,
        out_type=jax.ShapeDtypeStruct((NC, ACC_ROWS, D_FEAT), jnp.float32),
        mesh=_MESH,
        scratch_types=[
            pltpu.VMEM((nb, EB), jnp.int32),
            pltpu.VMEM((EB, D_FEAT), jnp.float32),
            pltpu.VMEM_SHARED((ACC_ROWS, D_FEAT), jnp.float32),
        ],
    )


NBUF = 2
SB = 8


def _spmm_body(nb0, nb1, g_hbm, rows_hbm, cols_hbm, zeros_hbm, out_hbm,
               rring, cring, bufs, acc, gsems, isem):
    cid = lax.axis_index("c")
    sid = lax.axis_index("s")
    wid = sid * NC + cid
    nb = jnp.where(cid == 0, nb0, nb1)
    ng = jnp.where(cid == 0, nb0 // SB, nb1 // SB)
    pltpu.sync_copy(zeros_hbm, acc.at[pl.ds(sid * STRIPE, STRIPE)])

    def stage(grp, slot, sem):
        off = pl.multiple_of(grp * SB, SB)
        roff = pl.multiple_of(slot * SB, SB)
        pltpu.async_copy(rows_hbm.at[wid, pl.ds(off, SB)],
                         rring.at[pl.ds(roff, SB)], sem)
        pltpu.async_copy(cols_hbm.at[wid, pl.ds(off, SB)],
                         cring.at[pl.ds(roff, SB)], sem)

    def stage_wait(grp, slot, sem):
        off = pl.multiple_of(grp * SB, SB)
        roff = pl.multiple_of(slot * SB, SB)
        pltpu.make_async_copy(rows_hbm.at[wid, pl.ds(off, SB)],
                              rring.at[pl.ds(roff, SB)], sem).wait()
        pltpu.make_async_copy(cols_hbm.at[wid, pl.ds(off, SB)],
                              cring.at[pl.ds(roff, SB)], sem).wait()

    stage(0, 0, isem)
    stage_wait(0, 0, isem)
    plsc.subcore_barrier()

    @pl.when(ng > 1)
    def _():
        stage(1, 1, isem)

    for j in range(NBUF):
        pltpu.async_copy(g_hbm.at[cring.at[j]], bufs[j], gsems[j])

    def group(i, carry):
        s = lax.rem(i, 2) * SB
        w = lax.rem(i + 1, 2) * SB

        @pl.when(i + 1 < ng)
        def _():
            stage_wait(i + 1, lax.rem(i + 1, 2), isem)

        for j in range(SB):
            b = SB * i + j
            k = j % NBUF
            nbase = s if j + NBUF < SB else w
            nj = (j + NBUF) % SB
            pltpu.make_async_copy(g_hbm.at[cring.at[s + j]], bufs[k],
                                  gsems[k]).wait()
            pltpu.sync_copy(bufs[k], acc.at[rring.at[s + j]], add=True)

            @pl.when(b + NBUF < nb)
            def _():
                pltpu.async_copy(g_hbm.at[cring.at[nbase + nj]], bufs[k],
                                 gsems[k])

        @pl.when(i + 2 < ng)
        def _():
            stage(i + 2, lax.rem(i, 2), isem)

        return carry

    lax.fori_loop(0, ng, group, 0)
    plsc.subcore_barrier()
    pltpu.sync_copy(acc.at[pl.ds(sid * STRIPE, STRIPE)],
                    out_hbm.at[cid, pl.ds(sid * STRIPE, STRIPE)])


def _make_spmm(nb0, nb1):
    return pl.kernel(
        functools.partial(_spmm_body, nb0, nb1),
        out_type=jax.ShapeDtypeStruct((NC, ACC_ROWS, D_FEAT), jnp.float32),
        mesh=_MESH,
        scratch_types=[
            pltpu.VMEM((2 * SB, EB), jnp.int32),
            pltpu.VMEM((2 * SB, EB), jnp.int32),
            [pltpu.VMEM((EB, D_FEAT), jnp.float32) for _ in range(NBUF)],
            pltpu.VMEM_SHARED((ACC_ROWS, D_FEAT), jnp.float32),
            [pltpu.SemaphoreType.DMA for _ in range(NBUF)],
            pltpu.SemaphoreType.DMA,
        ],
    )


def _prep_body(x_ref, degp_ref, g0_ref):
    x = x_ref[...]
    n = x.shape[0]
    mean = jnp.mean(x, axis=0, keepdims=True)
    xc = x - mean
    var = jnp.sum(xc * xc, axis=0, keepdims=True) / (n - 1)
    rstd = jnp.where(var > 0.0, lax.rsqrt(var), 1.0)
    deg = degp_ref[0, :N_NODES, 0:1] + degp_ref[1, :N_NODES, 0:1] + 1.0
    s = lax.rsqrt(deg)
    g0_ref[...] = xc * rstd * s


def _combine_body(zp_ref, g_ref, degp_ref, out_ref, *, last_hop):
    deg = degp_ref[0, :N_NODES, 0:1] + degp_ref[1, :N_NODES, 0:1] + 1.0
    scale = lax.rsqrt(deg) if last_hop else 1.0 / deg
    z = zp_ref[0, :N_NODES, :] + zp_ref[1, :N_NODES, :] + g_ref[...]
    out_ref[...] = z * scale


SHARE0 = 0.5


def _pack(arr, fill, nb_c, nb_max):
    pad = NS * nb_c * EB - arr.shape[0]
    if fill is None:
        filler = jnp.arange(pad, dtype=jnp.int32) % EB
    else:
        filler = jnp.full((pad,), fill, jnp.int32)
    p = jnp.concatenate([arr, filler])
    p = p.reshape(NS, nb_c, EB)
    if nb_max > nb_c:
        p = jnp.pad(p, ((0, 0), (0, nb_max - nb_c), (0, 0)))
    return p


def kernel(x, edge_index):
    e = edge_index.shape[1]
    row = edge_index[0].astype(jnp.int32)
    col = edge_index[1].astype(jnp.int32)
    onesd = jnp.ones((EB, D_FEAT), jnp.float32)
    zerosd = jnp.zeros((STRIPE, D_FEAT), jnp.float32)

    nbd = -(-e // (NW * EB))
    padd = NW * nbd * EB - e
    rows_d = jnp.concatenate(
        [row, jnp.full((padd,), DUMMY, jnp.int32)]).reshape(NW, nbd, EB)
    degp = _make_degree(nbd)(rows_d, onesd, zerosd)

    tb = -(-e // EB)
    nb0 = max(SB, ((int(tb * SHARE0 / NS) + SB // 2) // SB) * SB)
    e0 = min(NS * nb0 * EB, e)
    nb1 = -(-(e - e0) // (NS * EB)) if e > e0 else SB
    nb1 = max(SB, -(-nb1 // SB) * SB)
    nb_max = max(nb0, nb1)
    rows_p = jnp.stack(
        [_pack(row[:e0], DUMMY, nb0, nb_max),
         _pack(row[e0:], DUMMY, nb1, nb_max)],
        axis=1).reshape(NW, nb_max, EB)
    cols_p = jnp.stack(
        [_pack(col[:e0], None, nb0, nb_max),
         _pack(col[e0:], None, nb1, nb_max)],
        axis=1).reshape(NW, nb_max, EB)

    prep = pl.pallas_call(
        _prep_body,
        out_shape=jax.ShapeDtypeStruct((N_NODES, D_FEAT), jnp.float32),
    )
    g0 = prep(x, degp)

    spmm = _make_spmm(nb0, nb1)
    combine1 = pl.pallas_call(
        functools.partial(_combine_body, last_hop=False),
        out_shape=jax.ShapeDtypeStruct((N_NODES, D_FEAT), jnp.float32),
    )
    combine2 = pl.pallas_call(
        functools.partial(_combine_body, last_hop=True),
        out_shape=jax.ShapeDtypeStruct((N_NODES, D_FEAT), jnp.float32),
    )

    zp1 = spmm(g0, rows_p, cols_p, zerosd)
    g1 = combine1(zp1, g0, degp)
    zp2 = spmm(g1, rows_p, cols_p, zerosd)
    return combine2(zp2, g1, degp)
